# Initial kernel scaffold; baseline (speedup 1.0000x reference)
import jax
import jax.numpy as jnp
from jax.experimental import pallas as pl


def kernel(x, edge_index, W1, b1, W2, b2):
    raise NotImplementedError("write your pallas kernel here")


# speedup vs baseline: 7.2870x; 7.2870x over previous
import functools

import jax
import jax.numpy as jnp
from jax import lax
from jax.experimental import pallas as pl
from jax.experimental.pallas import tpu as pltpu
from jax.experimental.pallas import tpu_sc as plsc

_NC = 2
_NS = 16
_L = 16
_CH = 80
_L2 = 128


def _zero_rows(sh_ref, zbuf, start, count, zrows):
  for i in range(count // zrows):
    pltpu.sync_copy(zbuf, sh_ref.at[pl.ds(start + i * zrows, zrows)])
  rem = count % zrows
  if rem:
    pltpu.sync_copy(zbuf.at[pl.ds(0, rem)],
                    sh_ref.at[pl.ds(start + (count // zrows) * zrows, rem)])


def _sc_body(n_nodes, h_feat, e_edges, refs):
  (ylo, yhi, src_hbm, dst_hbm, olo, ohi,
   acc_sh, deg_sh, srcb, dstb, dstv0, dstv1, iotav,
   rows0, rows1, degt, gsem0, gsem1, osem0, osem1) = refs

  cid = lax.axis_index("c")
  sid = lax.axis_index("s")
  base_rows = ((n_nodes // _NS + _L2 - 1) // _L2) * _L2
  last_rows = n_nodes - (_NS - 1) * base_rows
  e_per_tile = e_edges // _NS
  n_chunks = e_per_tile // _CH
  one16 = jnp.ones((_L,), jnp.float32)

  nA = (n_chunks + 1) // 2
  nB = n_chunks - nA

  eb = pl.multiple_of(sid * e_per_tile, 8)
  pltpu.sync_copy(src_hbm.at[pl.ds(eb, nA * _CH)], srcb)
  pltpu.sync_copy(dst_hbm.at[pl.ds(eb, e_per_tile)], dstb)

  def zfill(r, carry):
    for kk in range(h_feat // _L):
      rows0[r, pl.ds(kk * _L, _L)] = jnp.zeros((_L,), jnp.float32)
      degt[r, pl.ds(kk * _L, _L)] = jnp.zeros((_L,), jnp.float32)
    return carry
  lax.fori_loop(0, _CH, zfill, 0)
  lane = plsc.cumsum(jnp.ones((_L,), jnp.int32)) - 1
  for kk in range(_CH // _L):
    iotav[pl.ds(kk * _L, _L)] = lane + jnp.int32(kk * _L)

  row0 = pl.multiple_of(sid * base_rows, 8)

  @pl.when(sid < _NS - 1)
  def _():
    _zero_rows(acc_sh, rows0, row0, base_rows, _CH)

  @pl.when(sid == _NS - 1)
  def _():
    _zero_rows(acc_sh, rows0, row0, last_rows, _CH)

  @pl.when(sid == 0)
  def _():
    pltpu.sync_copy(rows0, deg_sh)

  plsc.subcore_barrier()

  def fire_gather(j, rows_ref, sem):
    idx = srcb.at[pl.ds(pl.multiple_of(j * _CH, 8), _CH)]
    @pl.when(cid == 0)
    def _():
      pltpu.async_copy(ylo.at[idx], rows_ref, sem)
    @pl.when(cid == 1)
    def _():
      pltpu.async_copy(yhi.at[idx], rows_ref, sem)

  def wait_gather(rows_ref, sem):
    pltpu.make_async_copy(ylo.at[pl.ds(0, _CH)], rows_ref, sem).wait()

  def build_dstv(j, dstv):
    base = pl.multiple_of(j * _CH, 8)
    for kk in range(_CH // _L):
      d = dstb[pl.ds(base + kk * _L, _L)]
      dstv[pl.ds(kk * _L, _L)] = d
      plsc.addupdate_scatter(degt, [lax.shift_right_logical(d, 7),
                                    lax.bitwise_and(d, 127)], one16)

  def scatter(rows_ref, dstv):
    pltpu.sync_copy(rows_ref, acc_sh.at[dstv], add=True)

  def make_pair(goff):
    def pair(p, carry):
      j0 = 2 * p
      fire_gather(j0 + 1, rows1, gsem1)
      build_dstv(goff + j0, dstv0)
      wait_gather(rows0, gsem0)
      scatter(rows0, dstv0)
      fire_gather(j0 + 2, rows0, gsem0)
      build_dstv(goff + j0 + 1, dstv1)
      wait_gather(rows1, gsem1)
      scatter(rows1, dstv1)
      return carry
    return pair

  fire_gather(0, rows0, gsem0)
  lax.fori_loop(0, (nA - 1) // 2, make_pair(0), 0)
  build_dstv(nA - 1, dstv0)
  wait_gather(rows0, gsem0)
  scatter(rows0, dstv0)

  pltpu.sync_copy(src_hbm.at[pl.ds(pl.multiple_of(eb + nA * _CH, 8),
                                   nB * _CH)],
                  srcb.at[pl.ds(0, nB * _CH)])

  fire_gather(0, rows0, gsem0)
  lax.fori_loop(0, (nB - 2) // 2, make_pair(nA), 0)
  fire_gather(nB - 1, rows1, gsem1)
  build_dstv(nA + nB - 2, dstv0)
  wait_gather(rows0, gsem0)
  scatter(rows0, dstv0)
  build_dstv(nA + nB - 1, dstv1)
  wait_gather(rows1, gsem1)
  scatter(rows1, dstv1)

  pltpu.sync_copy(degt, deg_sh.at[iotav], add=True)
  plsc.subcore_barrier()
  pltpu.sync_copy(deg_sh, degt)

  out_ref = [olo, ohi]
  bufs = [rows0, rows1]
  osems = [osem0, osem1]

  def mean_out(start, nchunks):
    for i in range(nchunks):
      b = i % 2
      s0 = pl.multiple_of(start + i * _CH, 8)
      if i >= 2:
        pltpu.make_async_copy(bufs[b], olo.at[pl.ds(0, _CH)],
                              osems[b]).wait()
      pltpu.sync_copy(acc_sh.at[pl.ds(s0, _CH)], bufs[b])

      def grp(g, carry):
        nvec = s0 + g * _L + lane
        dvec = plsc.load_gather(degt, [lax.shift_right_logical(nvec, 7),
                                       lax.bitwise_and(nvec, 127)])
        ivec = 1.0 / jnp.maximum(dvec, 1.0)
        for r2 in range(_L):
          rbase = g * _L + r2
          iv = ivec[r2]
          for kk in range(h_feat // _L):
            sl = pl.ds(kk * _L, _L)
            bufs[b][rbase, sl] = bufs[b][rbase, sl] * iv
        return carry
      lax.fori_loop(0, _CH // _L, grp, 0)

      @pl.when(cid == 0)
      def _():
        pltpu.async_copy(bufs[b], olo.at[pl.ds(s0, _CH)], osems[b])
      @pl.when(cid == 1)
      def _():
        pltpu.async_copy(bufs[b], ohi.at[pl.ds(s0, _CH)], osems[b])
    for i in range(max(0, nchunks - 2), nchunks):
      pltpu.make_async_copy(bufs[i % 2], olo.at[pl.ds(0, _CH)],
                            osems[i % 2]).wait()

  @pl.when(sid < _NS - 1)
  def _():
    mean_out(row0, base_rows // _CH)

  @pl.when(sid == _NS - 1)
  def _():
    mean_out(row0, last_rows // _CH)


def _sc_segment_mean(y_lo, y_hi, src, dst, n_nodes):
  N, H = y_lo.shape
  E = src.shape[0]
  mesh = plsc.VectorSubcoreMesh(core_axis_name="c", subcore_axis_name="s")

  out_type = [jax.ShapeDtypeStruct((N, H), jnp.float32),
              jax.ShapeDtypeStruct((N, H), jnp.float32)]
  scratch = [
      pltpu.VMEM_SHARED((n_nodes, H), jnp.float32),
      pltpu.VMEM_SHARED((_CH, _L2), jnp.float32),
      pltpu.VMEM((((E // _NS // _CH + 1) // 2) * _CH,), jnp.int32),
      pltpu.VMEM((E // _NS,), jnp.int32),
      pltpu.VMEM((_CH,), jnp.int32),
      pltpu.VMEM((_CH,), jnp.int32),
      pltpu.VMEM((_CH,), jnp.int32),
      pltpu.VMEM((_CH, H), jnp.float32),
      pltpu.VMEM((_CH, H), jnp.float32),
      pltpu.VMEM((_CH, _L2), jnp.float32),
      pltpu.SemaphoreType.DMA,
      pltpu.SemaphoreType.DMA,
      pltpu.SemaphoreType.DMA,
      pltpu.SemaphoreType.DMA,
  ]

  body = functools.partial(_sc_body, n_nodes, H, E)

  def wrapped(*refs):
    body(refs)

  fn = pl.kernel(
      wrapped, mesh=mesh, out_type=out_type, scratch_types=scratch,
      compiler_params=pltpu.CompilerParams(needs_layout_passes=False))
  return fn(y_lo, y_hi, src, dst)


_BN = 400


def _mm_body(x_ref, wlo, whi, olo, ohi):
  olo[...] = jnp.dot(x_ref[...], wlo[...], preferred_element_type=jnp.float32)
  ohi[...] = jnp.dot(x_ref[...], whi[...], preferred_element_type=jnp.float32)


def _tc_matmul_split(x, W):
  N, D = x.shape
  H = D // 2
  nb = N // _BN
  out = jax.ShapeDtypeStruct((N, H), jnp.float32)
  return pl.pallas_call(
      _mm_body,
      grid=(nb,),
      in_specs=[
          pl.BlockSpec((_BN, D), lambda b: (b, 0)),
          pl.BlockSpec((D, H), lambda b: (0, 0)),
          pl.BlockSpec((D, H), lambda b: (0, 1)),
      ],
      out_specs=[pl.BlockSpec((_BN, H), lambda b: (b, 0)),
                 pl.BlockSpec((_BN, H), lambda b: (b, 0))],
      out_shape=[out, out],
  )(x, W, W)


def _mid_body(h_feat, m_lo, m_hi, y_lo, y_hi, b, w, olo, ohi):
  h_lo = jnp.maximum(m_lo[...] + y_lo[...] + b[0, :h_feat], 0.0)
  h_hi = jnp.maximum(m_hi[...] + y_hi[...] + b[0, h_feat:], 0.0)
  olo[...] = (
      jnp.dot(h_lo, w[:h_feat, :h_feat], preferred_element_type=jnp.float32)
      + jnp.dot(h_hi, w[h_feat:, :h_feat], preferred_element_type=jnp.float32))
  ohi[...] = (
      jnp.dot(h_lo, w[:h_feat, h_feat:], preferred_element_type=jnp.float32)
      + jnp.dot(h_hi, w[h_feat:, h_feat:], preferred_element_type=jnp.float32))


def _tc_mid(m_lo, m_hi, y_lo, y_hi, b, W):
  N, H = y_lo.shape
  D = W.shape[0]
  nb = N // _BN
  b2d = b.reshape(1, D)
  out = jax.ShapeDtypeStruct((N, H), jnp.float32)
  return pl.pallas_call(
      functools.partial(_mid_body, H),
      grid=(nb,),
      in_specs=[
          pl.BlockSpec((_BN, H), lambda b_: (b_, 0)),
          pl.BlockSpec((_BN, H), lambda b_: (b_, 0)),
          pl.BlockSpec((_BN, H), lambda b_: (b_, 0)),
          pl.BlockSpec((_BN, H), lambda b_: (b_, 0)),
          pl.BlockSpec((1, D), lambda b_: (0, 0)),
          pl.BlockSpec((D, D), lambda b_: (0, 0)),
      ],
      out_specs=[pl.BlockSpec((_BN, H), lambda b_: (b_, 0)),
                 pl.BlockSpec((_BN, H), lambda b_: (b_, 0))],
      out_shape=[out, out],
  )(m_lo, m_hi, y_lo, y_hi, b2d, W)


def _final_body(h_feat, m_lo, m_hi, y_lo, y_hi, b, out):
  lo = m_lo[...] + y_lo[...] + b[0, :h_feat]
  hi = m_hi[...] + y_hi[...] + b[0, h_feat:]
  out[...] = jnp.concatenate([lo, hi], axis=1)


def _tc_final(m_lo, m_hi, y_lo, y_hi, b):
  N, H = y_lo.shape
  D = b.shape[0]
  nb = N // _BN
  b2d = b.reshape(1, D)
  return pl.pallas_call(
      functools.partial(_final_body, H),
      grid=(nb,),
      in_specs=[
          pl.BlockSpec((_BN, H), lambda b_: (b_, 0)),
          pl.BlockSpec((_BN, H), lambda b_: (b_, 0)),
          pl.BlockSpec((_BN, H), lambda b_: (b_, 0)),
          pl.BlockSpec((_BN, H), lambda b_: (b_, 0)),
          pl.BlockSpec((1, D), lambda b_: (0, 0)),
      ],
      out_specs=pl.BlockSpec((_BN, D), lambda b_: (b_, 0)),
      out_shape=jax.ShapeDtypeStruct((N, D), jnp.float32),
  )(m_lo, m_hi, y_lo, y_hi, b2d)


def kernel(x, edge_index, W1, b1, W2, b2):
  N, D = x.shape
  src = edge_index[0]
  dst = edge_index[1]

  y1lo, y1hi = _tc_matmul_split(x, W1)
  m1lo, m1hi = _sc_segment_mean(y1lo, y1hi, src, dst, N)
  y2lo, y2hi = _tc_mid(m1lo, m1hi, y1lo, y1hi, b1, W2)
  m2lo, m2hi = _sc_segment_mean(y2lo, y2hi, src, dst, N)
  return _tc_final(m2lo, m2hi, y2lo, y2hi, b2)


# speedup vs baseline: 7.9986x; 1.0977x over previous
import functools

import jax
import jax.numpy as jnp
from jax import lax
from jax.experimental import pallas as pl
from jax.experimental.pallas import tpu as pltpu
from jax.experimental.pallas import tpu_sc as plsc

_NC = 2
_NS = 16
_L = 16
_CH = 80
_L2 = 128


def _zero_rows(sh_ref, zbuf, start, count, zrows):
  for i in range(count // zrows):
    pltpu.sync_copy(zbuf, sh_ref.at[pl.ds(start + i * zrows, zrows)])
  rem = count % zrows
  if rem:
    pltpu.sync_copy(zbuf.at[pl.ds(0, rem)],
                    sh_ref.at[pl.ds(start + (count // zrows) * zrows, rem)])


def _sc_body(n_nodes, h_feat, e_edges, refs):
  (ylo, yhi, src_hbm, dst_hbm, olo, ohi,
   acc_sh, deg_sh, srcb, dstb, dstv0, dstv1, iotav,
   rows0, rows1, degt, gsem0, gsem1, osem0, osem1) = refs

  cid = lax.axis_index("c")
  sid = lax.axis_index("s")
  base_rows = ((n_nodes // _NS + _L2 - 1) // _L2) * _L2
  last_rows = n_nodes - (_NS - 1) * base_rows
  e_per_tile = e_edges // _NS
  n_chunks = e_per_tile // _CH
  one16 = jnp.ones((_L,), jnp.float32)

  nA = (n_chunks + 1) // 2
  nB = n_chunks - nA

  eb = pl.multiple_of(sid * e_per_tile, 8)
  pltpu.sync_copy(src_hbm.at[pl.ds(eb, nA * _CH)], srcb)
  pltpu.sync_copy(dst_hbm.at[pl.ds(eb, e_per_tile)], dstb)

  def zfill(r, carry):
    for kk in range(h_feat // _L):
      rows0[r, pl.ds(kk * _L, _L)] = jnp.zeros((_L,), jnp.float32)
      degt[r, pl.ds(kk * _L, _L)] = jnp.zeros((_L,), jnp.float32)
    return carry
  lax.fori_loop(0, _CH, zfill, 0)
  lane = plsc.cumsum(jnp.ones((_L,), jnp.int32)) - 1
  for kk in range(_CH // _L):
    iotav[pl.ds(kk * _L, _L)] = lane + jnp.int32(kk * _L)

  row0 = pl.multiple_of(sid * base_rows, 8)

  @pl.when(sid < _NS - 1)
  def _():
    _zero_rows(acc_sh, rows0, row0, base_rows, _CH)

  @pl.when(sid == _NS - 1)
  def _():
    _zero_rows(acc_sh, rows0, row0, last_rows, _CH)

  @pl.when(sid == 0)
  def _():
    pltpu.sync_copy(rows0, deg_sh)

  plsc.subcore_barrier()

  def fire_gather(j, rows_ref, sem):
    idx = srcb.at[pl.ds(pl.multiple_of(j * _CH, 8), _CH)]
    @pl.when(cid == 0)
    def _():
      pltpu.async_copy(ylo.at[idx], rows_ref, sem)
    @pl.when(cid == 1)
    def _():
      pltpu.async_copy(yhi.at[idx], rows_ref, sem)

  def wait_gather(rows_ref, sem):
    pltpu.make_async_copy(ylo.at[pl.ds(0, _CH)], rows_ref, sem).wait()

  def build_dstv(j, dstv):
    base = pl.multiple_of(j * _CH, 8)
    for kk in range(_CH // _L):
      d = dstb[pl.ds(base + kk * _L, _L)]
      dstv[pl.ds(kk * _L, _L)] = d
      plsc.addupdate_scatter(degt, [lax.shift_right_logical(d, 7),
                                    lax.bitwise_and(d, 127)], one16)

  def scatter(rows_ref, dstv):
    pltpu.sync_copy(rows_ref, acc_sh.at[dstv], add=True)

  def make_pair(goff):
    def pair(p, carry):
      j0 = 2 * p
      fire_gather(j0 + 1, rows1, gsem1)
      build_dstv(goff + j0, dstv0)
      wait_gather(rows0, gsem0)
      scatter(rows0, dstv0)
      fire_gather(j0 + 2, rows0, gsem0)
      build_dstv(goff + j0 + 1, dstv1)
      wait_gather(rows1, gsem1)
      scatter(rows1, dstv1)
      return carry
    return pair

  fire_gather(0, rows0, gsem0)
  lax.fori_loop(0, (nA - 1) // 2, make_pair(0), 0)
  build_dstv(nA - 1, dstv0)
  wait_gather(rows0, gsem0)
  scatter(rows0, dstv0)

  pltpu.sync_copy(src_hbm.at[pl.ds(pl.multiple_of(eb + nA * _CH, 8),
                                   nB * _CH)],
                  srcb.at[pl.ds(0, nB * _CH)])

  fire_gather(0, rows0, gsem0)
  lax.fori_loop(0, (nB - 2) // 2, make_pair(nA), 0)
  fire_gather(nB - 1, rows1, gsem1)
  build_dstv(nA + nB - 2, dstv0)
  wait_gather(rows0, gsem0)
  scatter(rows0, dstv0)
  build_dstv(nA + nB - 1, dstv1)
  wait_gather(rows1, gsem1)
  scatter(rows1, dstv1)

  pltpu.sync_copy(degt, deg_sh.at[iotav], add=True)
  plsc.subcore_barrier()
  pltpu.sync_copy(deg_sh, degt)

  out_ref = [olo, ohi]
  bufs = [rows0, rows1]
  osems = [osem0, osem1]

  isems = [gsem0, gsem1]

  def fire_in(i, start):
    s0 = pl.multiple_of(start + i * _CH, 8)
    pltpu.async_copy(acc_sh.at[pl.ds(s0, _CH)], bufs[i % 2], isems[i % 2])

  def mean_out(start, nchunks):
    fire_in(0, start)
    for i in range(nchunks):
      b = i % 2
      s0 = pl.multiple_of(start + i * _CH, 8)
      if i + 1 < nchunks:
        if i >= 1:
          pltpu.make_async_copy(bufs[1 - b], olo.at[pl.ds(0, _CH)],
                                osems[1 - b]).wait()
        fire_in(i + 1, start)
      pltpu.make_async_copy(acc_sh.at[pl.ds(0, _CH)], bufs[b],
                            isems[b]).wait()

      def grp(g, carry):
        nvec = s0 + g * _L + lane
        dvec = plsc.load_gather(degt, [lax.shift_right_logical(nvec, 7),
                                       lax.bitwise_and(nvec, 127)])
        ivec = 1.0 / jnp.maximum(dvec, 1.0)
        for r2 in range(_L):
          rbase = g * _L + r2
          iv = ivec[r2]
          for kk in range(h_feat // _L):
            sl = pl.ds(kk * _L, _L)
            bufs[b][rbase, sl] = bufs[b][rbase, sl] * iv
        return carry
      lax.fori_loop(0, _CH // _L, grp, 0)

      @pl.when(cid == 0)
      def _():
        pltpu.async_copy(bufs[b], olo.at[pl.ds(s0, _CH)], osems[b])
      @pl.when(cid == 1)
      def _():
        pltpu.async_copy(bufs[b], ohi.at[pl.ds(s0, _CH)], osems[b])
    for i in range(max(0, nchunks - 2), nchunks):
      pltpu.make_async_copy(bufs[i % 2], olo.at[pl.ds(0, _CH)],
                            osems[i % 2]).wait()

  @pl.when(sid < _NS - 1)
  def _():
    mean_out(row0, base_rows // _CH)

  @pl.when(sid == _NS - 1)
  def _():
    mean_out(row0, last_rows // _CH)


def _sc_segment_mean(y_lo, y_hi, src, dst, n_nodes):
  N, H = y_lo.shape
  E = src.shape[0]
  mesh = plsc.VectorSubcoreMesh(core_axis_name="c", subcore_axis_name="s")

  out_type = [jax.ShapeDtypeStruct((N, H), jnp.float32),
              jax.ShapeDtypeStruct((N, H), jnp.float32)]
  scratch = [
      pltpu.VMEM_SHARED((n_nodes, H), jnp.float32),
      pltpu.VMEM_SHARED((_CH, _L2), jnp.float32),
      pltpu.VMEM((((E // _NS // _CH + 1) // 2) * _CH,), jnp.int32),
      pltpu.VMEM((E // _NS,), jnp.int32),
      pltpu.VMEM((_CH,), jnp.int32),
      pltpu.VMEM((_CH,), jnp.int32),
      pltpu.VMEM((_CH,), jnp.int32),
      pltpu.VMEM((_CH, H), jnp.float32),
      pltpu.VMEM((_CH, H), jnp.float32),
      pltpu.VMEM((_CH, _L2), jnp.float32),
      pltpu.SemaphoreType.DMA,
      pltpu.SemaphoreType.DMA,
      pltpu.SemaphoreType.DMA,
      pltpu.SemaphoreType.DMA,
  ]

  body = functools.partial(_sc_body, n_nodes, H, E)

  def wrapped(*refs):
    body(refs)

  fn = pl.kernel(
      wrapped, mesh=mesh, out_type=out_type, scratch_types=scratch,
      compiler_params=pltpu.CompilerParams(needs_layout_passes=False))
  return fn(y_lo, y_hi, src, dst)


_BN = 2000


def _mm_body(x_ref, wlo, whi, olo, ohi):
  olo[...] = jnp.dot(x_ref[...], wlo[...], preferred_element_type=jnp.float32)
  ohi[...] = jnp.dot(x_ref[...], whi[...], preferred_element_type=jnp.float32)


def _tc_matmul_split(x, W):
  N, D = x.shape
  H = D // 2
  nb = N // _BN
  out = jax.ShapeDtypeStruct((N, H), jnp.float32)
  return pl.pallas_call(
      _mm_body,
      grid=(nb,),
      in_specs=[
          pl.BlockSpec((_BN, D), lambda b: (b, 0)),
          pl.BlockSpec((D, H), lambda b: (0, 0)),
          pl.BlockSpec((D, H), lambda b: (0, 1)),
      ],
      out_specs=[pl.BlockSpec((_BN, H), lambda b: (b, 0)),
                 pl.BlockSpec((_BN, H), lambda b: (b, 0))],
      out_shape=[out, out],
  )(x, W, W)


def _mid_body(h_feat, m_lo, m_hi, y_lo, y_hi, b, w, olo, ohi):
  h_lo = jnp.maximum(m_lo[...] + y_lo[...] + b[0, :h_feat], 0.0)
  h_hi = jnp.maximum(m_hi[...] + y_hi[...] + b[0, h_feat:], 0.0)
  olo[...] = (
      jnp.dot(h_lo, w[:h_feat, :h_feat], preferred_element_type=jnp.float32)
      + jnp.dot(h_hi, w[h_feat:, :h_feat], preferred_element_type=jnp.float32))
  ohi[...] = (
      jnp.dot(h_lo, w[:h_feat, h_feat:], preferred_element_type=jnp.float32)
      + jnp.dot(h_hi, w[h_feat:, h_feat:], preferred_element_type=jnp.float32))


def _tc_mid(m_lo, m_hi, y_lo, y_hi, b, W):
  N, H = y_lo.shape
  D = W.shape[0]
  nb = N // _BN
  b2d = b.reshape(1, D)
  out = jax.ShapeDtypeStruct((N, H), jnp.float32)
  return pl.pallas_call(
      functools.partial(_mid_body, H),
      grid=(nb,),
      in_specs=[
          pl.BlockSpec((_BN, H), lambda b_: (b_, 0)),
          pl.BlockSpec((_BN, H), lambda b_: (b_, 0)),
          pl.BlockSpec((_BN, H), lambda b_: (b_, 0)),
          pl.BlockSpec((_BN, H), lambda b_: (b_, 0)),
          pl.BlockSpec((1, D), lambda b_: (0, 0)),
          pl.BlockSpec((D, D), lambda b_: (0, 0)),
      ],
      out_specs=[pl.BlockSpec((_BN, H), lambda b_: (b_, 0)),
                 pl.BlockSpec((_BN, H), lambda b_: (b_, 0))],
      out_shape=[out, out],
  )(m_lo, m_hi, y_lo, y_hi, b2d, W)


def _final_body(h_feat, m_lo, m_hi, y_lo, y_hi, b, out):
  lo = m_lo[...] + y_lo[...] + b[0, :h_feat]
  hi = m_hi[...] + y_hi[...] + b[0, h_feat:]
  out[...] = jnp.concatenate([lo, hi], axis=1)


def _tc_final(m_lo, m_hi, y_lo, y_hi, b):
  N, H = y_lo.shape
  D = b.shape[0]
  nb = N // _BN
  b2d = b.reshape(1, D)
  return pl.pallas_call(
      functools.partial(_final_body, H),
      grid=(nb,),
      in_specs=[
          pl.BlockSpec((_BN, H), lambda b_: (b_, 0)),
          pl.BlockSpec((_BN, H), lambda b_: (b_, 0)),
          pl.BlockSpec((_BN, H), lambda b_: (b_, 0)),
          pl.BlockSpec((_BN, H), lambda b_: (b_, 0)),
          pl.BlockSpec((1, D), lambda b_: (0, 0)),
      ],
      out_specs=pl.BlockSpec((_BN, D), lambda b_: (b_, 0)),
      out_shape=jax.ShapeDtypeStruct((N, D), jnp.float32),
  )(m_lo, m_hi, y_lo, y_hi, b2d)


def kernel(x, edge_index, W1, b1, W2, b2):
  N, D = x.shape
  src = edge_index[0]
  dst = edge_index[1]

  y1lo, y1hi = _tc_matmul_split(x, W1)
  m1lo, m1hi = _sc_segment_mean(y1lo, y1hi, src, dst, N)
  y2lo, y2hi = _tc_mid(m1lo, m1hi, y1lo, y1hi, b1, W2)
  m2lo, m2hi = _sc_segment_mean(y2lo, y2hi, src, dst, N)
  return _tc_final(m2lo, m2hi, y2lo, y2hi, b2)


# speedup vs baseline: 8.0797x; 1.0101x over previous
import functools

import jax
import jax.numpy as jnp
from jax import lax
from jax.experimental import pallas as pl
from jax.experimental.pallas import tpu as pltpu
from jax.experimental.pallas import tpu_sc as plsc

_NC = 2
_NS = 16
_L = 16
_CH = 80
_L2 = 128


def _zero_rows(sh_ref, zbuf, start, count, zrows):
  for i in range(count // zrows):
    pltpu.sync_copy(zbuf, sh_ref.at[pl.ds(start + i * zrows, zrows)])
  rem = count % zrows
  if rem:
    pltpu.sync_copy(zbuf.at[pl.ds(0, rem)],
                    sh_ref.at[pl.ds(start + (count // zrows) * zrows, rem)])


def _sc_body(n_nodes, h_feat, e_edges, refs):
  (ylo, yhi, src_hbm, dst_hbm, olo, ohi,
   acc_sh, deg_sh, srcb, dstb, dstv0, dstv1, iotav,
   rows0, rows1, degt, gsem0, gsem1, osem0, osem1) = refs

  cid = lax.axis_index("c")
  sid = lax.axis_index("s")
  base_rows = ((n_nodes // _NS + _L2 - 1) // _L2) * _L2
  last_rows = n_nodes - (_NS - 1) * base_rows
  e_per_tile = e_edges // _NS
  n_chunks = e_per_tile // _CH
  one16 = jnp.ones((_L,), jnp.float32)

  nA = (n_chunks + 1) // 2
  nB = n_chunks - nA

  eb = pl.multiple_of(sid * e_per_tile, 8)
  pltpu.async_copy(src_hbm.at[pl.ds(eb, nA * _CH)], srcb, osem0)
  pltpu.async_copy(dst_hbm.at[pl.ds(eb, e_per_tile)], dstb, osem1)

  def zfill(r, carry):
    for kk in range(h_feat // _L):
      rows0[r, pl.ds(kk * _L, _L)] = jnp.zeros((_L,), jnp.float32)
      degt[r, pl.ds(kk * _L, _L)] = jnp.zeros((_L,), jnp.float32)
    return carry
  lax.fori_loop(0, _CH, zfill, 0)
  lane = plsc.cumsum(jnp.ones((_L,), jnp.int32)) - 1
  for kk in range(_CH // _L):
    iotav[pl.ds(kk * _L, _L)] = lane + jnp.int32(kk * _L)

  row0 = pl.multiple_of(sid * base_rows, 8)

  @pl.when(sid < _NS - 1)
  def _():
    _zero_rows(acc_sh, rows0, row0, base_rows, _CH)

  @pl.when(sid == _NS - 1)
  def _():
    _zero_rows(acc_sh, rows0, row0, last_rows, _CH)

  @pl.when(sid == 0)
  def _():
    pltpu.sync_copy(rows0, deg_sh)

  pltpu.make_async_copy(src_hbm.at[pl.ds(eb, nA * _CH)], srcb, osem0).wait()
  pltpu.make_async_copy(dst_hbm.at[pl.ds(eb, e_per_tile)], dstb, osem1).wait()

  plsc.subcore_barrier()

  def fire_gather(j, rows_ref, sem):
    idx = srcb.at[pl.ds(pl.multiple_of(j * _CH, 8), _CH)]
    @pl.when(cid == 0)
    def _():
      pltpu.async_copy(ylo.at[idx], rows_ref, sem)
    @pl.when(cid == 1)
    def _():
      pltpu.async_copy(yhi.at[idx], rows_ref, sem)

  def wait_gather(rows_ref, sem):
    pltpu.make_async_copy(ylo.at[pl.ds(0, _CH)], rows_ref, sem).wait()

  def build_dstv(j, dstv):
    base = pl.multiple_of(j * _CH, 8)
    for kk in range(_CH // _L):
      d = dstb[pl.ds(base + kk * _L, _L)]
      dstv[pl.ds(kk * _L, _L)] = d
      plsc.addupdate_scatter(degt, [lax.shift_right_logical(d, 7),
                                    lax.bitwise_and(d, 127)], one16)

  def scatter(rows_ref, dstv):
    pltpu.sync_copy(rows_ref, acc_sh.at[dstv], add=True)

  def make_pair(goff):
    def pair(p, carry):
      j0 = 2 * p
      fire_gather(j0 + 1, rows1, gsem1)
      build_dstv(goff + j0, dstv0)
      wait_gather(rows0, gsem0)
      scatter(rows0, dstv0)
      fire_gather(j0 + 2, rows0, gsem0)
      build_dstv(goff + j0 + 1, dstv1)
      wait_gather(rows1, gsem1)
      scatter(rows1, dstv1)
      return carry
    return pair

  fire_gather(0, rows0, gsem0)
  lax.fori_loop(0, (nA - 1) // 2, make_pair(0), 0)
  build_dstv(nA - 1, dstv0)
  wait_gather(rows0, gsem0)
  scatter(rows0, dstv0)

  pltpu.sync_copy(src_hbm.at[pl.ds(pl.multiple_of(eb + nA * _CH, 8),
                                   nB * _CH)],
                  srcb.at[pl.ds(0, nB * _CH)])

  fire_gather(0, rows0, gsem0)
  lax.fori_loop(0, (nB - 2) // 2, make_pair(nA), 0)
  fire_gather(nB - 1, rows1, gsem1)
  build_dstv(nA + nB - 2, dstv0)
  wait_gather(rows0, gsem0)
  scatter(rows0, dstv0)
  build_dstv(nA + nB - 1, dstv1)
  wait_gather(rows1, gsem1)
  scatter(rows1, dstv1)

  pltpu.sync_copy(degt, deg_sh.at[iotav], add=True)
  plsc.subcore_barrier()
  pltpu.sync_copy(deg_sh, degt)

  out_ref = [olo, ohi]
  bufs = [rows0, rows1]
  osems = [osem0, osem1]

  isems = [gsem0, gsem1]

  def fire_in(i, start):
    s0 = pl.multiple_of(start + i * _CH, 8)
    pltpu.async_copy(acc_sh.at[pl.ds(s0, _CH)], bufs[i % 2], isems[i % 2])

  def mean_out(start, nchunks):
    fire_in(0, start)
    for i in range(nchunks):
      b = i % 2
      s0 = pl.multiple_of(start + i * _CH, 8)
      if i + 1 < nchunks:
        if i >= 1:
          pltpu.make_async_copy(bufs[1 - b], olo.at[pl.ds(0, _CH)],
                                osems[1 - b]).wait()
        fire_in(i + 1, start)
      pltpu.make_async_copy(acc_sh.at[pl.ds(0, _CH)], bufs[b],
                            isems[b]).wait()

      def grp(g, carry):
        nvec = s0 + g * _L + lane
        dvec = plsc.load_gather(degt, [lax.shift_right_logical(nvec, 7),
                                       lax.bitwise_and(nvec, 127)])
        ivec = 1.0 / jnp.maximum(dvec, 1.0)
        for r2 in range(_L):
          rbase = g * _L + r2
          iv = ivec[r2]
          for kk in range(h_feat // _L):
            sl = pl.ds(kk * _L, _L)
            bufs[b][rbase, sl] = bufs[b][rbase, sl] * iv
        return carry
      lax.fori_loop(0, _CH // _L, grp, 0)

      @pl.when(cid == 0)
      def _():
        pltpu.async_copy(bufs[b], olo.at[pl.ds(s0, _CH)], osems[b])
      @pl.when(cid == 1)
      def _():
        pltpu.async_copy(bufs[b], ohi.at[pl.ds(s0, _CH)], osems[b])
    for i in range(max(0, nchunks - 2), nchunks):
      pltpu.make_async_copy(bufs[i % 2], olo.at[pl.ds(0, _CH)],
                            osems[i % 2]).wait()

  @pl.when(sid < _NS - 1)
  def _():
    mean_out(row0, base_rows // _CH)

  @pl.when(sid == _NS - 1)
  def _():
    mean_out(row0, last_rows // _CH)


def _sc_segment_mean(y_lo, y_hi, src, dst, n_nodes):
  N, H = y_lo.shape
  E = src.shape[0]
  mesh = plsc.VectorSubcoreMesh(core_axis_name="c", subcore_axis_name="s")

  out_type = [jax.ShapeDtypeStruct((N, H), jnp.float32),
              jax.ShapeDtypeStruct((N, H), jnp.float32)]
  scratch = [
      pltpu.VMEM_SHARED((n_nodes, H), jnp.float32),
      pltpu.VMEM_SHARED((_CH, _L2), jnp.float32),
      pltpu.VMEM((((E // _NS // _CH + 1) // 2) * _CH,), jnp.int32),
      pltpu.VMEM((E // _NS,), jnp.int32),
      pltpu.VMEM((_CH,), jnp.int32),
      pltpu.VMEM((_CH,), jnp.int32),
      pltpu.VMEM((_CH,), jnp.int32),
      pltpu.VMEM((_CH, H), jnp.float32),
      pltpu.VMEM((_CH, H), jnp.float32),
      pltpu.VMEM((_CH, _L2), jnp.float32),
      pltpu.SemaphoreType.DMA,
      pltpu.SemaphoreType.DMA,
      pltpu.SemaphoreType.DMA,
      pltpu.SemaphoreType.DMA,
  ]

  body = functools.partial(_sc_body, n_nodes, H, E)

  def wrapped(*refs):
    body(refs)

  fn = pl.kernel(
      wrapped, mesh=mesh, out_type=out_type, scratch_types=scratch,
      compiler_params=pltpu.CompilerParams(needs_layout_passes=False))
  return fn(y_lo, y_hi, src, dst)


_BN = 2000


def _mm_body(x_ref, wlo, whi, olo, ohi):
  olo[...] = jnp.dot(x_ref[...], wlo[...], preferred_element_type=jnp.float32)
  ohi[...] = jnp.dot(x_ref[...], whi[...], preferred_element_type=jnp.float32)


def _tc_matmul_split(x, W):
  N, D = x.shape
  H = D // 2
  nb = N // _BN
  out = jax.ShapeDtypeStruct((N, H), jnp.float32)
  return pl.pallas_call(
      _mm_body,
      grid=(nb,),
      in_specs=[
          pl.BlockSpec((_BN, D), lambda b: (b, 0)),
          pl.BlockSpec((D, H), lambda b: (0, 0)),
          pl.BlockSpec((D, H), lambda b: (0, 1)),
      ],
      out_specs=[pl.BlockSpec((_BN, H), lambda b: (b, 0)),
                 pl.BlockSpec((_BN, H), lambda b: (b, 0))],
      out_shape=[out, out],
  )(x, W, W)


def _mid_body(h_feat, m_lo, m_hi, y_lo, y_hi, b, w, olo, ohi):
  h_lo = jnp.maximum(m_lo[...] + y_lo[...] + b[0, :h_feat], 0.0)
  h_hi = jnp.maximum(m_hi[...] + y_hi[...] + b[0, h_feat:], 0.0)
  olo[...] = (
      jnp.dot(h_lo, w[:h_feat, :h_feat], preferred_element_type=jnp.float32)
      + jnp.dot(h_hi, w[h_feat:, :h_feat], preferred_element_type=jnp.float32))
  ohi[...] = (
      jnp.dot(h_lo, w[:h_feat, h_feat:], preferred_element_type=jnp.float32)
      + jnp.dot(h_hi, w[h_feat:, h_feat:], preferred_element_type=jnp.float32))


def _tc_mid(m_lo, m_hi, y_lo, y_hi, b, W):
  N, H = y_lo.shape
  D = W.shape[0]
  nb = N // _BN
  b2d = b.reshape(1, D)
  out = jax.ShapeDtypeStruct((N, H), jnp.float32)
  return pl.pallas_call(
      functools.partial(_mid_body, H),
      grid=(nb,),
      in_specs=[
          pl.BlockSpec((_BN, H), lambda b_: (b_, 0)),
          pl.BlockSpec((_BN, H), lambda b_: (b_, 0)),
          pl.BlockSpec((_BN, H), lambda b_: (b_, 0)),
          pl.BlockSpec((_BN, H), lambda b_: (b_, 0)),
          pl.BlockSpec((1, D), lambda b_: (0, 0)),
          pl.BlockSpec((D, D), lambda b_: (0, 0)),
      ],
      out_specs=[pl.BlockSpec((_BN, H), lambda b_: (b_, 0)),
                 pl.BlockSpec((_BN, H), lambda b_: (b_, 0))],
      out_shape=[out, out],
  )(m_lo, m_hi, y_lo, y_hi, b2d, W)


def _final_body(h_feat, m_lo, m_hi, y_lo, y_hi, b, out):
  lo = m_lo[...] + y_lo[...] + b[0, :h_feat]
  hi = m_hi[...] + y_hi[...] + b[0, h_feat:]
  out[...] = jnp.concatenate([lo, hi], axis=1)


def _tc_final(m_lo, m_hi, y_lo, y_hi, b):
  N, H = y_lo.shape
  D = b.shape[0]
  nb = N // _BN
  b2d = b.reshape(1, D)
  return pl.pallas_call(
      functools.partial(_final_body, H),
      grid=(nb,),
      in_specs=[
          pl.BlockSpec((_BN, H), lambda b_: (b_, 0)),
          pl.BlockSpec((_BN, H), lambda b_: (b_, 0)),
          pl.BlockSpec((_BN, H), lambda b_: (b_, 0)),
          pl.BlockSpec((_BN, H), lambda b_: (b_, 0)),
          pl.BlockSpec((1, D), lambda b_: (0, 0)),
      ],
      out_specs=pl.BlockSpec((_BN, D), lambda b_: (b_, 0)),
      out_shape=jax.ShapeDtypeStruct((N, D), jnp.float32),
  )(m_lo, m_hi, y_lo, y_hi, b2d)


def kernel(x, edge_index, W1, b1, W2, b2):
  N, D = x.shape
  src = edge_index[0]
  dst = edge_index[1]

  y1lo, y1hi = _tc_matmul_split(x, W1)
  m1lo, m1hi = _sc_segment_mean(y1lo, y1hi, src, dst, N)
  y2lo, y2hi = _tc_mid(m1lo, m1hi, y1lo, y1hi, b1, W2)
  m2lo, m2hi = _sc_segment_mean(y2lo, y2hi, src, dst, N)
  return _tc_final(m2lo, m2hi, y2lo, y2hi, b2)
